# baseline (speedup 1.0000x reference)
import functools

import jax
import jax.numpy as jnp
from jax import lax
from jax.experimental import pallas as pl
from jax.experimental.pallas import tpu as pltpu
from jax.experimental.pallas import tpu_sc as plsc

_B, _L, _D = 64, 576, 768
_NC, _NS = 2, 16
_G, _J = 4, 8
_BPG = _B // _G
_RPW = _L // _J
_CHR = 8
_NCH = _RPW // _CHR
_KB = 8
_NBH = _BPG // _KB
_C4 = 4
_NCQ = _D // (16 * _C4)
_NOS = 2


def _sc_body(in_hbm, pos_hbm, cls_hbm, out_hbm,
             pos_v, i_v, o_v, sem_in, sem_out):
    wid = lax.axis_index("s") * _NC + lax.axis_index("c")
    g = wid // _J
    j = wid % _J
    b0 = g * _BPG
    r_base = j * _RPW

    def in_blk(rch, bh, pb):
        return pltpu.make_async_copy(
            in_hbm.at[pl.ds(b0 + bh * _KB, _KB),
                      pl.ds(r_base + rch * _CHR, _CHR)],
            i_v.at[pb], sem_in.at[pb])

    def out_row(rch, r, bh, q):
        return pltpu.make_async_copy(
            o_v.at[q],
            out_hbm.at[r_base + rch * _CHR + r, pl.ds(b0 + bh * _KB, _KB)],
            sem_out.at[q])

    in_blk(0, 0, 0).start()

    def chunk_body(rch, carry):
        for bh in range(_NBH):
            pb = bh
            if bh == 0:
                pltpu.sync_copy(
                    pos_hbm.at[pl.ds(r_base + rch * _CHR, _CHR)], pos_v)
            in_blk(rch, bh, pb).wait()
            if bh + 1 < _NBH:
                in_blk(rch, bh + 1, 1 - pb).start()
            else:
                @pl.when(rch + 1 < _NCH)
                def _(pb=pb):
                    in_blk(rch + 1, 0, 1 - pb).start()

            for r in range(_CHR):
                q = r % _NOS
                if r >= _NOS:
                    out_row(rch, r - _NOS, bh, q).wait()
                elif bh > 0:
                    out_row(rch, _CHR - _NOS + r, bh - 1, q).wait()
                else:
                    @pl.when(rch > 0)
                    def _(r=r, q=q):
                        out_row(rch - 1, _CHR - _NOS + r, _NBH - 1, q).wait()

                def cq_body(cq, c3, r=r, q=q, pb=pb):
                    for c4 in range(_C4):
                        off = cq * (16 * _C4) + c4 * 16
                        p = pos_v[r, pl.ds(off, 16)]
                        for k in range(_KB):
                            o_v[q, k, pl.ds(off, 16)] = (
                                i_v[pb, k, r, pl.ds(off, 16)] + p
                            )
                    return c3

                lax.fori_loop(0, _NCQ, cq_body, 0)
                out_row(rch, r, bh, q).start()
        return carry

    lax.fori_loop(0, _NCH, chunk_body, 0)

    for r in range(_CHR - _NOS, _CHR):
        out_row(_NCH - 1, r, _NBH - 1, r % _NOS).wait()

    @pl.when(j == 0)
    def _():
        pltpu.sync_copy(cls_hbm.at[0], o_v.at[1, pl.ds(0, 1)])
        pltpu.sync_copy(pos_hbm.at[pl.ds(_L, 1)], o_v.at[1, pl.ds(1, 1)])
        for c in range(_D // 16):
            off = c * 16
            p = o_v[1, 0, pl.ds(off, 16)] + o_v[1, 1, pl.ds(off, 16)]
            for k in range(_KB):
                o_v[0, k, pl.ds(off, 16)] = p
        for bh in range(_NBH):
            pltpu.sync_copy(o_v.at[0],
                            out_hbm.at[_L, pl.ds(b0 + bh * _KB, _KB)])


@functools.partial(
    pl.kernel,
    mesh=plsc.VectorSubcoreMesh(core_axis_name="c", subcore_axis_name="s"),
    out_type=jax.ShapeDtypeStruct((_L + 1, _B, _D), jnp.float32),
    scratch_types=[
        pltpu.VMEM((_CHR, _D), jnp.float32),
        pltpu.VMEM((2, _KB, _CHR, _D), jnp.float32),
        pltpu.VMEM((_NOS, _KB, _D), jnp.float32),
        pltpu.SemaphoreType.DMA((2,)),
        pltpu.SemaphoreType.DMA((_NOS,)),
    ],
)
def _sc_kernel(in_hbm, pos_hbm, cls_hbm, out_hbm,
               pos_v, i_v, o_v, sem_in, sem_out):
    _sc_body(in_hbm, pos_hbm, cls_hbm, out_hbm,
             pos_v, i_v, o_v, sem_in, sem_out)


def kernel(inputs, pos_table, class_token):
    out_t = _sc_kernel(inputs, pos_table, class_token)
    return jnp.transpose(out_t, (1, 0, 2))


# speedup vs baseline: 2.7685x; 2.7685x over previous
import functools

import jax
import jax.numpy as jnp
from jax import lax
from jax.experimental import pallas as pl
from jax.experimental.pallas import tpu as pltpu
from jax.experimental.pallas import tpu_sc as plsc

_B, _L, _D = 64, 576, 768
_NC, _NS = 2, 16
_G, _J = 4, 8
_BPG = _B // _G
_RPW = _L // _J
_CHR = 8
_NCH = _RPW // _CHR
_KB = 8
_NBH = _BPG // _KB


def _sc_body(in_hbm, pos_hbm, cls_hbm, out_hbm, pos_v, i_v, sem_in, sem_out):
    wid = lax.axis_index("s") * _NC + lax.axis_index("c")
    g = wid // _J
    j = wid % _J
    b0 = g * _BPG
    r_base = j * _RPW

    def in_blk(rch, bh, pb):
        return pltpu.make_async_copy(
            in_hbm.at[pl.ds(b0 + bh * _KB, _KB),
                      pl.ds(r_base + rch * _CHR, _CHR)],
            i_v.at[pb], sem_in.at[pb])

    def out_row(rch, r, bh, pb):
        return pltpu.make_async_copy(
            i_v.at[pb, pl.ds(0, _KB), r],
            out_hbm.at[r_base + rch * _CHR + r, pl.ds(b0 + bh * _KB, _KB)],
            sem_out.at[pb])

    in_blk(0, 0, 0).start()

    def chunk_body(rch, carry):
        for bh in range(_NBH):
            pb = bh
            if bh == 0:
                pltpu.sync_copy(
                    pos_hbm.at[pl.ds(r_base + rch * _CHR, _CHR)], pos_v)
            in_blk(rch, bh, pb).wait()
            if bh + 1 < _NBH:
                for r in range(_CHR):
                    @pl.when(rch > 0)
                    def _(r=r, pb=pb):
                        out_row(rch, r, bh, 1 - pb).wait()
                in_blk(rch, bh + 1, 1 - pb).start()
            else:
                @pl.when(rch + 1 < _NCH)
                def _(pb=pb, bh=bh):
                    for r in range(_CHR):
                        out_row(rch, r, bh, 1 - pb).wait()
                    in_blk(rch + 1, 0, 1 - pb).start()

            def row_body(r, c3, bh=bh, pb=pb):
                for c in range(_D // 16):
                    off = c * 16
                    p = pos_v[r, pl.ds(off, 16)]
                    for k in range(_KB):
                        i_v[pb, k, r, pl.ds(off, 16)] = (
                            i_v[pb, k, r, pl.ds(off, 16)] + p
                        )
                out_row(rch, r, bh, pb).start()
                return c3

            lax.fori_loop(0, _CHR, row_body, 0)
        return carry

    lax.fori_loop(0, _NCH, chunk_body, 0)

    for pb in range(2):
        for r in range(_CHR):
            out_row(_NCH - 1, r, pb, pb).wait()

    @pl.when(j == 0)
    def _():
        pltpu.sync_copy(cls_hbm.at[0], i_v.at[1, 0, pl.ds(0, 1)])
        pltpu.sync_copy(pos_hbm.at[pl.ds(_L, 1)], i_v.at[1, 0, pl.ds(1, 1)])
        for c in range(_D // 16):
            off = c * 16
            p = i_v[1, 0, 0, pl.ds(off, 16)] + i_v[1, 0, 1, pl.ds(off, 16)]
            for k in range(_KB):
                i_v[0, 0, k, pl.ds(off, 16)] = p
        for bh in range(_NBH):
            pltpu.sync_copy(i_v.at[0, 0],
                            out_hbm.at[_L, pl.ds(b0 + bh * _KB, _KB)])


@functools.partial(
    pl.kernel,
    mesh=plsc.VectorSubcoreMesh(core_axis_name="c", subcore_axis_name="s"),
    out_type=jax.ShapeDtypeStruct((_L + 1, _B, _D), jnp.float32),
    scratch_types=[
        pltpu.VMEM((_CHR, _D), jnp.float32),
        pltpu.VMEM((2, _KB, _CHR, _D), jnp.float32),
        pltpu.SemaphoreType.DMA((2,)),
        pltpu.SemaphoreType.DMA((2,)),
    ],
)
def _sc_kernel(in_hbm, pos_hbm, cls_hbm, out_hbm, pos_v, i_v, sem_in, sem_out):
    _sc_body(in_hbm, pos_hbm, cls_hbm, out_hbm, pos_v, i_v, sem_in, sem_out)


def kernel(inputs, pos_table, class_token):
    out_t = _sc_kernel(inputs, pos_table, class_token)
    return jnp.transpose(out_t, (1, 0, 2))
